# initial kernel scaffold (unmeasured)
import jax
import jax.numpy as jnp
from jax import lax
from jax.experimental import pallas as pl
from jax.experimental.pallas import tpu as pltpu

N_DEV = 4
SQ = 2048
SKV_SH = 2048
HQ = 8
DH = 128
D = HQ * DH
BLK = 64
SCALE = 0.08838834764831843
NEG = -1e9
QC = 512


def kernel(x, Wq, K_ext, V_ext, Wo):
    x2 = x[0]
    K = K_ext[0]
    V = V_ext[0]

    def body(x_ref, wq_ref, k_ref, v_ref, wo_ref, out_ref,
             q_ref, acc_ref, ml_ref, ctx_comm, ml_comm,
             ctx_ssem, ctx_rsem, ml_ssem, ml_rsem):
        my = lax.axis_index("i")
        left = lax.rem(my + N_DEV - 1, N_DEV)
        right = lax.rem(my + 1, N_DEV)

        barrier_sem = pltpu.get_barrier_semaphore()
        for nbr in (left, right):
            pl.semaphore_signal(
                barrier_sem, inc=1,
                device_id=(nbr,), device_id_type=pl.DeviceIdType.MESH,
            )
        pl.semaphore_wait(barrier_sem, 2)

        q_ref[:, :] = jnp.dot(
            x_ref[:, :], wq_ref[:, :], preferred_element_type=jnp.float32
        ) * SCALE

        kb_off = my * (SKV_SH // BLK)
        for h in range(HQ):
            kh = k_ref[:, h, :]
            vh = v_ref[:, h, :]
            for c in range(SQ // QC):
                qh = q_ref[c * QC:(c + 1) * QC, h * DH:(h + 1) * DH]
                s = lax.dot_general(
                    qh, kh, (((1,), (1,)), ((), ())),
                    preferred_element_type=jnp.float32,
                )
                rb = (lax.broadcasted_iota(jnp.int32, (QC, SKV_SH), 0)
                      + c * QC) // BLK
                cb = lax.broadcasted_iota(jnp.int32, (QC, SKV_SH), 1) // BLK \
                    + kb_off
                mask = (rb == cb) | (cb == 0) | (lax.rem(rb + cb, 3) == 0)
                s = jnp.where(mask, s, NEG)
                m = jnp.max(s, axis=1, keepdims=True)
                w = jnp.exp(s - m)
                l = jnp.sum(w, axis=1, keepdims=True)
                ctx = jnp.dot(w, vh, preferred_element_type=jnp.float32)
                sl = pl.ds(c * QC, QC)
                acc_ref[h, sl, :] = ctx
                ctx_comm[0, h, sl, :] = ctx
                ml_ref[sl, h:h + 1] = m
                ml_ref[sl, HQ + h:HQ + h + 1] = l
                ml_comm[0, sl, h:h + 1] = m
                ml_comm[0, sl, HQ + h:HQ + h + 1] = l

        for hop in range(N_DEV - 1):
            ss, rs = hop % 2, (hop + 1) % 2
            r_ctx = pltpu.make_async_remote_copy(
                src_ref=ctx_comm.at[ss], dst_ref=ctx_comm.at[rs],
                send_sem=ctx_ssem.at[ss], recv_sem=ctx_rsem.at[rs],
                device_id=(right,), device_id_type=pl.DeviceIdType.MESH,
            )
            r_ml = pltpu.make_async_remote_copy(
                src_ref=ml_comm.at[ss], dst_ref=ml_comm.at[rs],
                send_sem=ml_ssem.at[ss], recv_sem=ml_rsem.at[rs],
                device_id=(right,), device_id_type=pl.DeviceIdType.MESH,
            )
            r_ctx.start()
            r_ml.start()
            r_ctx.wait()
            r_ml.wait()
            for h in range(HQ):
                m_a = ml_ref[:, h:h + 1]
                l_a = ml_ref[:, HQ + h:HQ + h + 1]
                m_b = ml_comm[rs, :, h:h + 1]
                l_b = ml_comm[rs, :, HQ + h:HQ + h + 1]
                m_n = jnp.maximum(m_a, m_b)
                ea = jnp.exp(m_a - m_n)
                eb = jnp.exp(m_b - m_n)
                acc_ref[h, :, :] = ea * acc_ref[h, :, :] \
                    + eb * ctx_comm[rs, h, :, :]
                ml_ref[:, h:h + 1] = m_n
                ml_ref[:, HQ + h:HQ + h + 1] = ea * l_a + eb * l_b

        for h in range(HQ):
            q_ref[:, h * DH:(h + 1) * DH] = \
                acc_ref[h, :, :] / ml_ref[:, HQ + h:HQ + h + 1]
        out_ref[:, :] = jnp.dot(
            q_ref[:, :], wo_ref[:, :], preferred_element_type=jnp.float32
        )

    out = pl.pallas_call(
        body,
        out_shape=jax.ShapeDtypeStruct((SQ, D), jnp.float32),
        in_specs=[pl.BlockSpec(memory_space=pltpu.VMEM)] * 5,
        out_specs=pl.BlockSpec(memory_space=pltpu.VMEM),
        scratch_shapes=[
            pltpu.VMEM((SQ, D), jnp.float32),
            pltpu.VMEM((HQ, SQ, DH), jnp.float32),
            pltpu.VMEM((SQ, 2 * HQ), jnp.float32),
            pltpu.VMEM((2, HQ, SQ, DH), jnp.float32),
            pltpu.VMEM((2, SQ, 2 * HQ), jnp.float32),
            pltpu.SemaphoreType.DMA((2,)),
            pltpu.SemaphoreType.DMA((2,)),
            pltpu.SemaphoreType.DMA((2,)),
            pltpu.SemaphoreType.DMA((2,)),
        ],
        compiler_params=pltpu.CompilerParams(collective_id=0),
    )(x2, Wq, K, V, Wo)
    return out[None, :, :]


# baseline (device time: 578182 ns/iter reference)
import jax
import jax.numpy as jnp
from jax import lax
from jax.experimental import pallas as pl
from jax.experimental.pallas import tpu as pltpu

N_DEV = 4
SQ = 2048
SKV_SH = 2048
HQ = 8
DH = 128
D = HQ * DH
BLK = 64
SCALE = 0.08838834764831843
NEG = -1e9
QC = 256


def _local_attn(x2, Wq, K, V):

    def body(x_ref, wq_ref, k_ref, v_ref, ctx_ref, m_ref, l_ref):
        my = lax.axis_index("i")
        qc = pl.program_id(0)
        h = pl.program_id(1)
        kb_off = my * (SKV_SH // BLK)

        q = jnp.dot(x_ref[:, :], wq_ref[:, :],
                    preferred_element_type=jnp.float32) * SCALE
        k = k_ref[0, :, :]
        v = v_ref[0, :, :]
        s = lax.dot_general(q, k, (((1,), (1,)), ((), ())),
                            preferred_element_type=jnp.float32)
        rb = (lax.broadcasted_iota(jnp.int32, (QC, SKV_SH), 0)
              + qc * QC) // BLK
        cb = lax.broadcasted_iota(jnp.int32, (QC, SKV_SH), 1) // BLK + kb_off
        mask = (rb == cb) | (cb == 0) | (lax.rem(rb + cb, 3) == 0)
        s = jnp.where(mask, s, NEG)
        m = jnp.max(s, axis=1, keepdims=True)
        w = jnp.exp(s - m)
        l = jnp.sum(w, axis=1, keepdims=True)
        ctx_ref[:, :] = jnp.dot(w, v, preferred_element_type=jnp.float32)
        col = lax.broadcasted_iota(jnp.int32, (QC, HQ), 1)
        m_ref[:, :] = jnp.where(col == h, m, m_ref[:, :])
        l_ref[:, :] = jnp.where(col == h, l, l_ref[:, :])

    return pl.pallas_call(
        body,
        grid=(SQ // QC, HQ),
        in_specs=[
            pl.BlockSpec((QC, D), lambda qc, h: (qc, 0)),
            pl.BlockSpec((D, DH), lambda qc, h: (0, h)),
            pl.BlockSpec((1, SKV_SH, DH), lambda qc, h: (h, 0, 0)),
            pl.BlockSpec((1, SKV_SH, DH), lambda qc, h: (h, 0, 0)),
        ],
        out_specs=[
            pl.BlockSpec((QC, DH), lambda qc, h: (qc, h)),
            pl.BlockSpec((QC, HQ), lambda qc, h: (qc, 0)),
            pl.BlockSpec((QC, HQ), lambda qc, h: (qc, 0)),
        ],
        out_shape=[
            jax.ShapeDtypeStruct((SQ, D), jnp.float32),
            jax.ShapeDtypeStruct((SQ, HQ), jnp.float32),
            jax.ShapeDtypeStruct((SQ, HQ), jnp.float32),
        ],
    )(x2, Wq, K, V)


def _ring_merge(ctx_p, m_p, l_p):

    def body(ctx_in, m_in, l_in, out_ref,
             ml_ref, ctx_comm, ml_comm,
             ctx_ssem, ctx_rsem, ml_ssem, ml_rsem):
        my = lax.axis_index("i")
        left = lax.rem(my + N_DEV - 1, N_DEV)
        right = lax.rem(my + 1, N_DEV)

        barrier_sem = pltpu.get_barrier_semaphore()
        for nbr in (left, right):
            pl.semaphore_signal(
                barrier_sem, inc=1,
                device_id=(nbr,), device_id_type=pl.DeviceIdType.MESH,
            )
        pl.semaphore_wait(barrier_sem, 2)

        for h in range(HQ):
            cds = pl.ds(h * DH, DH)
            out_ref[:, cds] = ctx_in[:, cds]
            ctx_comm[0, :, cds] = ctx_in[:, cds]
        ml_ref[:, 0:HQ] = m_in[:, :]
        ml_ref[:, HQ:2 * HQ] = l_in[:, :]
        ml_comm[0, :, 0:HQ] = m_in[:, :]
        ml_comm[0, :, HQ:2 * HQ] = l_in[:, :]

        for hop in range(N_DEV - 1):
            ss, rs = hop % 2, (hop + 1) % 2
            r_ctx = pltpu.make_async_remote_copy(
                src_ref=ctx_comm.at[ss], dst_ref=ctx_comm.at[rs],
                send_sem=ctx_ssem.at[ss], recv_sem=ctx_rsem.at[rs],
                device_id=(right,), device_id_type=pl.DeviceIdType.MESH,
            )
            r_ml = pltpu.make_async_remote_copy(
                src_ref=ml_comm.at[ss], dst_ref=ml_comm.at[rs],
                send_sem=ml_ssem.at[ss], recv_sem=ml_rsem.at[rs],
                device_id=(right,), device_id_type=pl.DeviceIdType.MESH,
            )
            r_ctx.start()
            r_ml.start()
            r_ctx.wait()
            r_ml.wait()

            for h in range(HQ):
                cds = pl.ds(h * DH, DH)
                m_a = ml_ref[:, h:h + 1]
                l_a = ml_ref[:, HQ + h:HQ + h + 1]
                m_b = ml_comm[rs, :, h:h + 1]
                l_b = ml_comm[rs, :, HQ + h:HQ + h + 1]
                m_n = jnp.maximum(m_a, m_b)
                ea = jnp.exp(m_a - m_n)
                eb = jnp.exp(m_b - m_n)
                out_ref[:, cds] = ea * out_ref[:, cds] + eb * ctx_comm[rs, :, cds]
                ml_ref[:, h:h + 1] = m_n
                ml_ref[:, HQ + h:HQ + h + 1] = ea * l_a + eb * l_b

        for h in range(HQ):
            cds = pl.ds(h * DH, DH)
            out_ref[:, cds] = out_ref[:, cds] / ml_ref[:, HQ + h:HQ + h + 1]

    return pl.pallas_call(
        body,
        out_shape=jax.ShapeDtypeStruct((SQ, D), jnp.float32),
        in_specs=[pl.BlockSpec(memory_space=pltpu.VMEM)] * 3,
        out_specs=pl.BlockSpec(memory_space=pltpu.VMEM),
        scratch_shapes=[
            pltpu.VMEM((SQ, 2 * HQ), jnp.float32),
            pltpu.VMEM((2, SQ, D), jnp.float32),
            pltpu.VMEM((2, SQ, 2 * HQ), jnp.float32),
            pltpu.SemaphoreType.DMA((2,)),
            pltpu.SemaphoreType.DMA((2,)),
            pltpu.SemaphoreType.DMA((2,)),
            pltpu.SemaphoreType.DMA((2,)),
        ],
        compiler_params=pltpu.CompilerParams(
            collective_id=0, vmem_limit_bytes=60 * 1024 * 1024,
        ),
    )(ctx_p, m_p, l_p)


def _out_proj(ctx, Wo):
    QB = 512

    def body(c_ref, wo_ref, o_ref):
        o_ref[:, :] = jnp.dot(c_ref[:, :], wo_ref[:, :],
                              preferred_element_type=jnp.float32)

    return pl.pallas_call(
        body,
        grid=(SQ // QB,),
        in_specs=[
            pl.BlockSpec((QB, D), lambda i: (i, 0)),
            pl.BlockSpec((D, D), lambda i: (0, 0)),
        ],
        out_specs=pl.BlockSpec((QB, D), lambda i: (i, 0)),
        out_shape=jax.ShapeDtypeStruct((SQ, D), jnp.float32),
    )(ctx, Wo)


def kernel(x, Wq, K_ext, V_ext, Wo):
    x2 = x[0]
    K = K_ext[0].transpose(1, 0, 2)
    V = V_ext[0].transpose(1, 0, 2)
    ctx_p, m_p, l_p = _local_attn(x2, Wq, K, V)
    ctx = _ring_merge(ctx_p, m_p, l_p)
    out = _out_proj(ctx, Wo)
    return out[None, :, :]


# device time: 361177 ns/iter; 1.6008x vs baseline; 1.6008x over previous
import jax
import jax.numpy as jnp
from jax import lax
from jax.experimental import pallas as pl
from jax.experimental.pallas import tpu as pltpu

N_DEV = 4
SQ = 2048
SKV_SH = 2048
HQ = 8
DH = 128
D = HQ * DH
BLK = 64
SCALE = 0.08838834764831843
NEG = -1e9
QC = 256
QR = SQ // N_DEV


def _local_attn(x2, Wq, K, V):

    def body(x_ref, wq_ref, k_ref, v_ref, ctx_ref, m_ref, l_ref):
        my = lax.axis_index("i")
        qc = pl.program_id(0)
        h = pl.program_id(1)
        kb_off = my * (SKV_SH // BLK)

        q = jnp.dot(x_ref[:, :], wq_ref[:, :],
                    preferred_element_type=jnp.float32) * SCALE
        k = k_ref[0, :, :]
        v = v_ref[0, :, :]
        s = lax.dot_general(q, k, (((1,), (1,)), ((), ())),
                            preferred_element_type=jnp.float32)
        rb = (lax.broadcasted_iota(jnp.int32, (QC, SKV_SH), 0)
              + qc * QC) // BLK
        cb = lax.broadcasted_iota(jnp.int32, (QC, SKV_SH), 1) // BLK + kb_off
        mask = (rb == cb) | (cb == 0) | (lax.rem(rb + cb, 3) == 0)
        s = jnp.where(mask, s, NEG)
        m = jnp.max(s, axis=1, keepdims=True)
        w = jnp.exp(s - m)
        l = jnp.sum(w, axis=1, keepdims=True)
        ctx_ref[:, :] = jnp.dot(w, v, preferred_element_type=jnp.float32)
        col = lax.broadcasted_iota(jnp.int32, (QC, HQ), 1)
        m_ref[:, :] = jnp.where(col == h, m, m_ref[:, :])
        l_ref[:, :] = jnp.where(col == h, l, l_ref[:, :])

    return pl.pallas_call(
        body,
        grid=(SQ // QC, HQ),
        in_specs=[
            pl.BlockSpec((QC, D), lambda qc, h: (qc, 0)),
            pl.BlockSpec((D, DH), lambda qc, h: (0, h)),
            pl.BlockSpec((1, SKV_SH, DH), lambda qc, h: (h, 0, 0)),
            pl.BlockSpec((1, SKV_SH, DH), lambda qc, h: (h, 0, 0)),
        ],
        out_specs=[
            pl.BlockSpec((QC, DH), lambda qc, h: (qc, h)),
            pl.BlockSpec((QC, HQ), lambda qc, h: (qc, 0)),
            pl.BlockSpec((QC, HQ), lambda qc, h: (qc, 0)),
        ],
        out_shape=[
            jax.ShapeDtypeStruct((SQ, D), jnp.float32),
            jax.ShapeDtypeStruct((SQ, HQ), jnp.float32),
            jax.ShapeDtypeStruct((SQ, HQ), jnp.float32),
        ],
    )(x2, Wq, K, V)


def _ring_merge(ctx_p, m_p, l_p):

    def body(ctx_in, m_in, l_in, out_ref,
             acc_ref, ml_ref, ctx_comm, ml_comm,
             ctx_ssem, ctx_rsem, ml_ssem, ml_rsem,
             ag_ssem, ag_rsem, loc_sem):
        my = lax.axis_index("i")
        left = lax.rem(my + N_DEV - 1, N_DEV)
        right = lax.rem(my + 1, N_DEV)
        opp = lax.rem(my + 2, N_DEV)

        barrier_sem = pltpu.get_barrier_semaphore()
        for nbr in (left, right):
            pl.semaphore_signal(
                barrier_sem, inc=1,
                device_id=(nbr,), device_id_type=pl.DeviceIdType.MESH,
            )
        pl.semaphore_wait(barrier_sem, 2)

        for q in range(N_DEV):
            rds = pl.ds(q * QR, QR)
            for h in range(HQ):
                cds = pl.ds(h * DH, DH)
                acc_ref[q, :, cds] = ctx_in[rds, cds]
            ml_ref[q, :, 0:HQ] = m_in[rds, :]
            ml_ref[q, :, HQ:2 * HQ] = l_in[rds, :]

        for s in range(N_DEV - 1):
            slot = s % 2
            sq = lax.rem(my - s + N_DEV, N_DEV)
            rq = lax.rem(my - 1 - s + 2 * N_DEV, N_DEV)
            r_ctx = pltpu.make_async_remote_copy(
                src_ref=acc_ref.at[sq], dst_ref=ctx_comm.at[slot],
                send_sem=ctx_ssem.at[slot], recv_sem=ctx_rsem.at[slot],
                device_id=(right,), device_id_type=pl.DeviceIdType.MESH,
            )
            r_ml = pltpu.make_async_remote_copy(
                src_ref=ml_ref.at[sq], dst_ref=ml_comm.at[slot],
                send_sem=ml_ssem.at[slot], recv_sem=ml_rsem.at[slot],
                device_id=(right,), device_id_type=pl.DeviceIdType.MESH,
            )
            r_ctx.start()
            r_ml.start()
            r_ctx.wait()
            r_ml.wait()
            for h in range(HQ):
                cds = pl.ds(h * DH, DH)
                m_a = ml_ref[rq, :, h:h + 1]
                l_a = ml_ref[rq, :, HQ + h:HQ + h + 1]
                m_b = ml_comm[slot, :, h:h + 1]
                l_b = ml_comm[slot, :, HQ + h:HQ + h + 1]
                m_n = jnp.maximum(m_a, m_b)
                ea = jnp.exp(m_a - m_n)
                eb = jnp.exp(m_b - m_n)
                acc_ref[rq, :, cds] = (ea * acc_ref[rq, :, cds]
                                       + eb * ctx_comm[slot, :, cds])
                ml_ref[rq, :, h:h + 1] = m_n
                ml_ref[rq, :, HQ + h:HQ + h + 1] = ea * l_a + eb * l_b

        oq = lax.rem(my + 1, N_DEV)
        for h in range(HQ):
            cds = pl.ds(h * DH, DH)
            acc_ref[oq, :, cds] = (acc_ref[oq, :, cds]
                                   / ml_ref[oq, :, HQ + h:HQ + h + 1])

        sends = []
        for i, peer in enumerate((right, opp, left)):
            r = pltpu.make_async_remote_copy(
                src_ref=acc_ref.at[oq],
                dst_ref=out_ref.at[pl.ds(oq * QR, QR), :],
                send_sem=ag_ssem.at[i], recv_sem=ag_rsem.at[i],
                device_id=(peer,), device_id_type=pl.DeviceIdType.MESH,
            )
            r.start()
            sends.append(r)
        loc = pltpu.make_async_copy(
            acc_ref.at[oq], out_ref.at[pl.ds(oq * QR, QR), :], loc_sem,
        )
        loc.start()
        for slot, q in ((0, lax.rem(my, N_DEV)),
                        (1, lax.rem(my + 3, N_DEV)),
                        (2, lax.rem(my + 2, N_DEV))):
            r = pltpu.make_async_remote_copy(
                src_ref=acc_ref.at[0],
                dst_ref=out_ref.at[pl.ds(q * QR, QR), :],
                send_sem=ag_ssem.at[0],
                recv_sem=ag_rsem.at[slot],
                device_id=(left,), device_id_type=pl.DeviceIdType.MESH,
            )
            r.wait_recv()
        for r in sends:
            r.wait_send()
        loc.wait()

    return pl.pallas_call(
        body,
        out_shape=jax.ShapeDtypeStruct((SQ, D), jnp.float32),
        in_specs=[pl.BlockSpec(memory_space=pltpu.VMEM)] * 3,
        out_specs=pl.BlockSpec(memory_space=pltpu.VMEM),
        scratch_shapes=[
            pltpu.VMEM((N_DEV, QR, D), jnp.float32),
            pltpu.VMEM((N_DEV, QR, 2 * HQ), jnp.float32),
            pltpu.VMEM((2, QR, D), jnp.float32),
            pltpu.VMEM((2, QR, 2 * HQ), jnp.float32),
            pltpu.SemaphoreType.DMA((2,)),
            pltpu.SemaphoreType.DMA((2,)),
            pltpu.SemaphoreType.DMA((2,)),
            pltpu.SemaphoreType.DMA((2,)),
            pltpu.SemaphoreType.DMA((3,)),
            pltpu.SemaphoreType.DMA((3,)),
            pltpu.SemaphoreType.DMA,
        ],
        compiler_params=pltpu.CompilerParams(
            collective_id=0, vmem_limit_bytes=60 * 1024 * 1024,
        ),
    )(ctx_p, m_p, l_p)


def _out_proj(ctx, Wo):
    QB = 512

    def body(c_ref, wo_ref, o_ref):
        o_ref[:, :] = jnp.dot(c_ref[:, :], wo_ref[:, :],
                              preferred_element_type=jnp.float32)

    return pl.pallas_call(
        body,
        grid=(SQ // QB,),
        in_specs=[
            pl.BlockSpec((QB, D), lambda i: (i, 0)),
            pl.BlockSpec((D, D), lambda i: (0, 0)),
        ],
        out_specs=pl.BlockSpec((QB, D), lambda i: (i, 0)),
        out_shape=jax.ShapeDtypeStruct((SQ, D), jnp.float32),
    )(ctx, Wo)


def kernel(x, Wq, K_ext, V_ext, Wo):
    x2 = x[0]
    K = K_ext[0].transpose(1, 0, 2)
    V = V_ext[0].transpose(1, 0, 2)
    ctx_p, m_p, l_p = _local_attn(x2, Wq, K, V)
    ctx = _ring_merge(ctx_p, m_p, l_p)
    out = _out_proj(ctx, Wo)
    return out[None, :, :]


# device time: 232595 ns/iter; 2.4858x vs baseline; 1.5528x over previous
import jax
import jax.numpy as jnp
from jax import lax
from jax.experimental import pallas as pl
from jax.experimental.pallas import tpu as pltpu

N_DEV = 4
SQ = 2048
SKV_SH = 2048
HQ = 8
DH = 128
D = HQ * DH
BLK = 64
SCALE = 0.08838834764831843
NEG = -1e9
QC = 256
QR = SQ // N_DEV


NCB = 11
CW = NCB * BLK


def _local_attn(x2, Wq, K, V):
    NLB = SKV_SH // BLK

    def body(x_ref, wq_ref, kc_ref, vc_ref, k_ref, v_ref,
             ctx_ref, m_ref, l_ref):
        my = lax.axis_index("i")
        qb = pl.program_id(0)

        q_all = jnp.dot(x_ref[:, :], wq_ref[:, :],
                        preferred_element_type=jnp.float32) * SCALE

        c_star = lax.rem(3 - lax.rem(qb + my * NLB, 3), 3)
        colc = lax.broadcasted_iota(jnp.int32, (1, CW), 1)
        pad_ok = (c_star != 2) | (colc < (NCB - 1) * BLK)

        lb_d = qb - my * NLB
        in_rng = (lb_d >= 0) & (lb_d < NLB)
        safe_lb = jnp.where(in_rng, lb_d, 0)
        valid_d = in_rng & (lax.rem(safe_lb, 3) != c_star)
        valid_0 = (my == 0) & (c_star != 0)

        for h in range(HQ):
            q = q_all[:, h * DH:(h + 1) * DH]
            kc = kc_ref[c_star, h, :, :]
            vc = vc_ref[c_star, h, :, :]
            s_c = lax.dot_general(q, kc, (((1,), (1,)), ((), ())),
                                  preferred_element_type=jnp.float32)
            s_c = jnp.where(pad_ok, s_c, NEG)

            kx = k_ref[h, pl.ds(safe_lb * BLK, BLK), :]
            vx = v_ref[h, pl.ds(safe_lb * BLK, BLK), :]
            k0 = k_ref[h, 0:BLK, :]
            v0 = v_ref[h, 0:BLK, :]
            s_d = lax.dot_general(q, kx, (((1,), (1,)), ((), ())),
                                  preferred_element_type=jnp.float32)
            s_0 = lax.dot_general(q, k0, (((1,), (1,)), ((), ())),
                                  preferred_element_type=jnp.float32)
            s_d = jnp.where(valid_d, s_d, NEG)
            s_0 = jnp.where(valid_0, s_0, NEG)

            m = jnp.maximum(
                jnp.max(s_c, axis=1, keepdims=True),
                jnp.maximum(jnp.max(s_d, axis=1, keepdims=True),
                            jnp.max(s_0, axis=1, keepdims=True)),
            )
            w_c = jnp.exp(s_c - m)
            w_d = jnp.exp(s_d - m)
            w_0 = jnp.exp(s_0 - m)
            l = (jnp.sum(w_c, axis=1, keepdims=True)
                 + jnp.sum(w_d, axis=1, keepdims=True)
                 + jnp.sum(w_0, axis=1, keepdims=True))
            ctx_ref[:, h * DH:(h + 1) * DH] = (
                jnp.dot(w_c, vc, preferred_element_type=jnp.float32)
                + jnp.dot(w_d, vx, preferred_element_type=jnp.float32)
                + jnp.dot(w_0, v0, preferred_element_type=jnp.float32)
            )
            m_ref[:, h:h + 1] = m
            l_ref[:, h:h + 1] = l

    return pl.pallas_call(
        body,
        grid=(SQ // BLK,),
        in_specs=[
            pl.BlockSpec((BLK, D), lambda qb: (qb, 0)),
            pl.BlockSpec(memory_space=pltpu.VMEM),
            pl.BlockSpec(memory_space=pltpu.VMEM),
            pl.BlockSpec(memory_space=pltpu.VMEM),
            pl.BlockSpec(memory_space=pltpu.VMEM),
            pl.BlockSpec(memory_space=pltpu.VMEM),
        ],
        out_specs=[
            pl.BlockSpec((BLK, D), lambda qb: (qb, 0)),
            pl.BlockSpec((BLK, HQ), lambda qb: (qb, 0)),
            pl.BlockSpec((BLK, HQ), lambda qb: (qb, 0)),
        ],
        out_shape=[
            jax.ShapeDtypeStruct((SQ, D), jnp.float32),
            jax.ShapeDtypeStruct((SQ, HQ), jnp.float32),
            jax.ShapeDtypeStruct((SQ, HQ), jnp.float32),
        ],
        compiler_params=pltpu.CompilerParams(
            vmem_limit_bytes=60 * 1024 * 1024,
        ),
    )(x2, Wq, *_class_gather(K), *_class_gather(V), K, V)


def _class_gather(t):
    nlb = SKV_SH // BLK
    tb = t.reshape(HQ, nlb, BLK, DH)
    cls = []
    for c in range(3):
        idx = [lb for lb in range(nlb) if lb % 3 == c]
        while len(idx) < NCB:
            idx.append(idx[-1])
        cls.append(tb[:, idx].reshape(HQ, CW, DH))
    return (jnp.stack(cls, axis=0),)


def _ring_merge(ctx_p, m_p, l_p):

    BF = jnp.bfloat16

    def body(ctx_in, m_in, l_in, out_ref,
             ctx_send, ml_ref, acc_q, ml_q, ag_send,
             ctx_comm, ml_comm,
             ctx_ssem, ctx_rsem, ml_ssem, ml_rsem,
             ag_ssem, ag_rsem, loc_sem):
        my = lax.axis_index("i")
        left = lax.rem(my + N_DEV - 1, N_DEV)
        right = lax.rem(my + 1, N_DEV)
        opp = lax.rem(my + 2, N_DEV)

        barrier_sem = pltpu.get_barrier_semaphore()
        for nbr in (left, right, opp):
            pl.semaphore_signal(
                barrier_sem, inc=1,
                device_id=(nbr,), device_id_type=pl.DeviceIdType.MESH,
            )
        pl.semaphore_wait(barrier_sem, 3)

        for q in range(N_DEV):
            rds = pl.ds(q * QR, QR)
            for h in range(HQ):
                cds = pl.ds(h * DH, DH)
                ctx_send[q, :, cds] = ctx_in[rds, cds].astype(BF)
            ml_ref[q, :, 0:HQ] = m_in[rds, :]
            ml_ref[q, :, HQ:2 * HQ] = l_in[rds, :]

        rs_sends = []
        for j, peer in enumerate((right, opp, left)):
            pq = lax.rem(my + 2 + j, N_DEV)
            r_ctx = pltpu.make_async_remote_copy(
                src_ref=ctx_send.at[pq], dst_ref=ctx_comm.at[j],
                send_sem=ctx_ssem.at[j], recv_sem=ctx_rsem.at[j],
                device_id=(peer,), device_id_type=pl.DeviceIdType.MESH,
            )
            r_ml = pltpu.make_async_remote_copy(
                src_ref=ml_ref.at[pq], dst_ref=ml_comm.at[j],
                send_sem=ml_ssem.at[j], recv_sem=ml_rsem.at[j],
                device_id=(peer,), device_id_type=pl.DeviceIdType.MESH,
            )
            r_ctx.start()
            r_ml.start()
            rs_sends.extend((r_ctx, r_ml))

        oq = lax.rem(my + 1, N_DEV)
        loc0 = pltpu.make_async_copy(
            ctx_in.at[pl.ds(oq * QR, QR), :], acc_q, loc_sem,
        )
        loc0.start()
        ml_q[:, :] = ml_ref[oq, :, :]
        loc0.wait()

        for j in range(3):
            r_ctx = pltpu.make_async_remote_copy(
                src_ref=ctx_send.at[0], dst_ref=ctx_comm.at[j],
                send_sem=ctx_ssem.at[j], recv_sem=ctx_rsem.at[j],
                device_id=(left,), device_id_type=pl.DeviceIdType.MESH,
            )
            r_ml = pltpu.make_async_remote_copy(
                src_ref=ml_ref.at[0], dst_ref=ml_comm.at[j],
                send_sem=ml_ssem.at[j], recv_sem=ml_rsem.at[j],
                device_id=(left,), device_id_type=pl.DeviceIdType.MESH,
            )
            r_ctx.wait_recv()
            r_ml.wait_recv()
            for h in range(HQ):
                cds = pl.ds(h * DH, DH)
                m_a = ml_q[:, h:h + 1]
                l_a = ml_q[:, HQ + h:HQ + h + 1]
                m_b = ml_comm[j, :, h:h + 1]
                l_b = ml_comm[j, :, HQ + h:HQ + h + 1]
                m_n = jnp.maximum(m_a, m_b)
                ea = jnp.exp(m_a - m_n)
                eb = jnp.exp(m_b - m_n)
                acc_q[:, cds] = (ea * acc_q[:, cds]
                                 + eb * ctx_comm[j, :, cds].astype(jnp.float32))
                ml_q[:, h:h + 1] = m_n
                ml_q[:, HQ + h:HQ + h + 1] = ea * l_a + eb * l_b
        for r in rs_sends:
            r.wait_send()

        for h in range(HQ):
            cds = pl.ds(h * DH, DH)
            ag_send[:, cds] = (acc_q[:, cds]
                               / ml_q[:, HQ + h:HQ + h + 1]).astype(BF)

        sends = []
        for i, peer in enumerate((right, opp, left)):
            r = pltpu.make_async_remote_copy(
                src_ref=ag_send,
                dst_ref=out_ref.at[pl.ds(oq * QR, QR), :],
                send_sem=ag_ssem.at[i], recv_sem=ag_rsem.at[i],
                device_id=(peer,), device_id_type=pl.DeviceIdType.MESH,
            )
            r.start()
            sends.append(r)
        loc = pltpu.make_async_copy(
            ag_send, out_ref.at[pl.ds(oq * QR, QR), :], loc_sem,
        )
        loc.start()
        for slot, q in ((0, lax.rem(my, N_DEV)),
                        (1, lax.rem(my + 3, N_DEV)),
                        (2, lax.rem(my + 2, N_DEV))):
            r = pltpu.make_async_remote_copy(
                src_ref=ag_send,
                dst_ref=out_ref.at[pl.ds(q * QR, QR), :],
                send_sem=ag_ssem.at[0],
                recv_sem=ag_rsem.at[slot],
                device_id=(left,), device_id_type=pl.DeviceIdType.MESH,
            )
            r.wait_recv()
        for r in sends:
            r.wait_send()
        loc.wait()

    return pl.pallas_call(
        body,
        out_shape=jax.ShapeDtypeStruct((SQ, D), BF),
        in_specs=[pl.BlockSpec(memory_space=pltpu.VMEM)] * 3,
        out_specs=pl.BlockSpec(memory_space=pltpu.VMEM),
        scratch_shapes=[
            pltpu.VMEM((N_DEV, QR, D), BF),
            pltpu.VMEM((N_DEV, QR, 2 * HQ), jnp.float32),
            pltpu.VMEM((QR, D), jnp.float32),
            pltpu.VMEM((QR, 2 * HQ), jnp.float32),
            pltpu.VMEM((QR, D), BF),
            pltpu.VMEM((3, QR, D), BF),
            pltpu.VMEM((3, QR, 2 * HQ), jnp.float32),
            pltpu.SemaphoreType.DMA((3,)),
            pltpu.SemaphoreType.DMA((3,)),
            pltpu.SemaphoreType.DMA((3,)),
            pltpu.SemaphoreType.DMA((3,)),
            pltpu.SemaphoreType.DMA((3,)),
            pltpu.SemaphoreType.DMA((3,)),
            pltpu.SemaphoreType.DMA,
        ],
        compiler_params=pltpu.CompilerParams(
            collective_id=0, vmem_limit_bytes=60 * 1024 * 1024,
        ),
    )(ctx_p, m_p, l_p)


def _out_proj(ctx, Wo):
    QB = 512

    def body(c_ref, wo_ref, o_ref):
        o_ref[:, :] = jnp.dot(c_ref[:, :], wo_ref[:, :],
                              preferred_element_type=jnp.float32)

    return pl.pallas_call(
        body,
        grid=(SQ // QB,),
        in_specs=[
            pl.BlockSpec((QB, D), lambda i: (i, 0)),
            pl.BlockSpec((D, D), lambda i: (0, 0)),
        ],
        out_specs=pl.BlockSpec((QB, D), lambda i: (i, 0)),
        out_shape=jax.ShapeDtypeStruct((SQ, D), jnp.float32),
    )(ctx, Wo)


def kernel(x, Wq, K_ext, V_ext, Wo):
    x2 = x[0]
    K = K_ext[0].transpose(1, 0, 2)
    V = V_ext[0].transpose(1, 0, 2)
    ctx_p, m_p, l_p = _local_attn(x2, Wq, K, V)
    ctx = _ring_merge(ctx_p, m_p, l_p)
    out = _out_proj(ctx, Wo)
    return out[None, :, :]
